# Initial kernel scaffold; baseline (speedup 1.0000x reference)
import jax
import jax.numpy as jnp
from jax.experimental import pallas as pl


def kernel(x, pos, word_table, pos_table):
    raise NotImplementedError("write your pallas kernel here")


# speedup vs baseline: 1.0868x; 1.0868x over previous
import functools

import jax
import jax.numpy as jnp
from jax import lax
from jax.experimental import pallas as pl
from jax.experimental.pallas import tpu as pltpu
from jax.experimental.pallas import tpu_sc as plsc

B, S = 4096, 200
N = B * S
WD = 60
PD = 4
D = WD + PD
L = 16
VW = 24
KPW = 3
NW = 32
ROWS_PER_W = N // NW
CHUNK = 128
NCHUNK = ROWS_PER_W // CHUNK
NI = KPW * CHUNK


def _emb_body(qi_hbm, pv_hbm, ov_hbm, wt_hbm, pt_hbm, out_hbm,
              qi_v, pi_v, ov_v, win_v, ptv_v, comb_v, sem):
    wid = lax.axis_index("s") * 2 + lax.axis_index("c")
    base0 = wid * ROWS_PER_W
    pltpu.sync_copy(pt_hbm, ptv_v)
    io = lax.iota(jnp.int32, L)
    msk = io < (L - PD)

    def chunk_body(i, carry):
        base = base0 + i * CHUNK
        pltpu.sync_copy(qi_hbm.at[pl.ds(KPW * base, NI)], qi_v)
        pltpu.sync_copy(pv_hbm.at[pl.ds(base, CHUNK)], pi_v)
        pltpu.sync_copy(ov_hbm.at[pl.ds(base, CHUNK)], ov_v)
        pltpu.async_copy(wt_hbm.at[qi_v], win_v, sem).wait()

        def blk(b, c2):
            xo = ov_v[pl.ds(b * L, L)]
            pv = pi_v[pl.ds(b * L, L)]
            for j in range(L):
                r = b * L + j
                off = xo[j]
                for m in range(3):
                    comb_v[pl.ds(D * r + m * L, L)] = \
                        win_v[KPW * r, pl.ds(off + m * L, L)]
                w3 = win_v[KPW * r, pl.ds(off + 3 * L, L)]
                comb_v[pl.ds(D * r + 3 * L, L)] = \
                    jnp.where(msk, w3, ptv_v[pv[j], :])
            return c2

        lax.fori_loop(0, CHUNK // L, blk, 0)
        pltpu.sync_copy(comb_v, out_hbm.at[pl.ds(D * base, D * CHUNK)])
        return carry

    lax.fori_loop(0, NCHUNK, chunk_body, 0)


def kernel(x, pos, word_table, pos_table):
    xf = x.reshape(N)
    pf = pos.reshape(N)
    wt24 = word_table.reshape(word_table.shape[0] * WD // VW, VW)
    q = (5 * xf) >> 1
    qi = jnp.stack([q, q + 1, q + 2], axis=-1).reshape(KPW * N)
    ov = (PD * KPW) * (xf & 1)
    pt_pad = jnp.zeros((pos_table.shape[0], L), pos_table.dtype)
    pt_pad = lax.dynamic_update_slice(pt_pad, pos_table, (0, L - PD))
    mesh = plsc.VectorSubcoreMesh(core_axis_name="c", subcore_axis_name="s")
    run = functools.partial(
        pl.kernel,
        mesh=mesh,
        compiler_params=pltpu.CompilerParams(use_tc_tiling_on_sc=False),
        out_type=jax.ShapeDtypeStruct((N * D,), jnp.float32),
        scratch_types=[
            pltpu.VMEM((NI,), jnp.int32),
            pltpu.VMEM((CHUNK,), jnp.int32),
            pltpu.VMEM((CHUNK,), jnp.int32),
            pltpu.VMEM((NI, VW), jnp.float32),
            pltpu.VMEM((pos_table.shape[0], L), jnp.float32),
            pltpu.VMEM((CHUNK * D,), jnp.float32),
            pltpu.SemaphoreType.DMA,
        ],
    )(_emb_body)
    out = run(qi, pf, ov, wt24, pt_pad)
    return out.reshape(B, S, D)


# speedup vs baseline: 1.2445x; 1.1451x over previous
import functools

import jax
import jax.numpy as jnp
from jax import lax
from jax.experimental import pallas as pl
from jax.experimental.pallas import tpu as pltpu
from jax.experimental.pallas import tpu_sc as plsc

B, S = 4096, 200
N = B * S
WD = 60
PD = 4
D = WD + PD
L = 16
VW = 24
KPW = 3
NW = 32
ROWS_PER_W = N // NW
CHUNK = 256
NCHUNK = ROWS_PER_W // CHUNK
NPAIR = NCHUNK // 2
NI = KPW * CHUNK


def _emb_body(qi_hbm, pv_hbm, ov_hbm, wt_hbm, pt_hbm, out_hbm,
              qi0, qi1, pi0, pi1, ov0, ov1, win0, win1, comb0, comb1,
              ptv_v, sg0, sg1, so0, so1):
    wid = lax.axis_index("s") * 2 + lax.axis_index("c")
    base0 = wid * ROWS_PER_W
    pltpu.sync_copy(pt_hbm, ptv_v)
    io = lax.iota(jnp.int32, L)
    msk = io < (L - PD)

    qis = (qi0, qi1)
    pis = (pi0, pi1)
    ovs = (ov0, ov1)
    wins = (win0, win1)
    combs = (comb0, comb1)
    sgs = (sg0, sg1)
    sos = (so0, so1)

    def load_idx(i, k):
        base = base0 + i * CHUNK
        pltpu.sync_copy(qi_hbm.at[pl.ds(KPW * base, NI)], qis[k])
        pltpu.sync_copy(pv_hbm.at[pl.ds(base, CHUNK)], pis[k])
        pltpu.sync_copy(ov_hbm.at[pl.ds(base, CHUNK)], ovs[k])

    def start_gather(k):
        return pltpu.async_copy(wt_hbm.at[qis[k]], wins[k], sgs[k])

    def wait_gather(k):
        pltpu.make_async_copy(wt_hbm.at[qis[k]], wins[k], sgs[k]).wait()

    def assemble(k):
        win_v = wins[k]
        comb_v = combs[k]

        def blk(b, c2):
            xo = ovs[k][pl.ds(b * L, L)]
            pv = pis[k][pl.ds(b * L, L)]
            for j in range(L):
                r = b * L + j
                off = xo[j]
                for m in range(3):
                    comb_v[pl.ds(D * r + m * L, L)] = \
                        win_v[KPW * r, pl.ds(off + m * L, L)]
                w3 = win_v[KPW * r, pl.ds(off + 3 * L, L)]
                comb_v[pl.ds(D * r + 3 * L, L)] = \
                    jnp.where(msk, w3, ptv_v[pv[j], :])
            return c2

        lax.fori_loop(0, CHUNK // L, blk, 0)

    def start_out(i, k):
        base = base0 + i * CHUNK
        return pltpu.async_copy(
            combs[k], out_hbm.at[pl.ds(D * base, D * CHUNK)], sos[k])

    def wait_out(i, k):
        base = base0 + i * CHUNK
        pltpu.make_async_copy(
            combs[k], out_hbm.at[pl.ds(D * base, D * CHUNK)], sos[k]).wait()

    load_idx(0, 0)
    start_gather(0)

    def pair(p, carry):
        for b in range(2):
            i = 2 * p + b
            nk = (b + 1) % 2
            @pl.when(i + 1 < NCHUNK)
            def _():
                load_idx(i + 1, nk)
                start_gather(nk)

            wait_gather(b)

            @pl.when(p > 0)
            def _():
                wait_out(i - 2, b)

            assemble(b)
            start_out(i, b)
        return carry

    lax.fori_loop(0, NPAIR, pair, 0)
    wait_out(NCHUNK - 2, 0)
    wait_out(NCHUNK - 1, 1)


def kernel(x, pos, word_table, pos_table):
    xf = x.reshape(N)
    pf = pos.reshape(N)
    wt24 = word_table.reshape(word_table.shape[0] * WD // VW, VW)
    q = (5 * xf) >> 1
    qi = jnp.stack([q, q + 1, q + 2], axis=-1).reshape(KPW * N)
    ov = (PD * KPW) * (xf & 1)
    pt_pad = jnp.zeros((pos_table.shape[0], L), pos_table.dtype)
    pt_pad = lax.dynamic_update_slice(pt_pad, pos_table, (0, L - PD))
    mesh = plsc.VectorSubcoreMesh(core_axis_name="c", subcore_axis_name="s")
    run = functools.partial(
        pl.kernel,
        mesh=mesh,
        compiler_params=pltpu.CompilerParams(use_tc_tiling_on_sc=False),
        out_type=jax.ShapeDtypeStruct((N * D,), jnp.float32),
        scratch_types=[
            pltpu.VMEM((NI,), jnp.int32),
            pltpu.VMEM((NI,), jnp.int32),
            pltpu.VMEM((CHUNK,), jnp.int32),
            pltpu.VMEM((CHUNK,), jnp.int32),
            pltpu.VMEM((CHUNK,), jnp.int32),
            pltpu.VMEM((CHUNK,), jnp.int32),
            pltpu.VMEM((NI, VW), jnp.float32),
            pltpu.VMEM((NI, VW), jnp.float32),
            pltpu.VMEM((CHUNK * D,), jnp.float32),
            pltpu.VMEM((CHUNK * D,), jnp.float32),
            pltpu.VMEM((pos_table.shape[0], L), jnp.float32),
            pltpu.SemaphoreType.DMA,
            pltpu.SemaphoreType.DMA,
            pltpu.SemaphoreType.DMA,
            pltpu.SemaphoreType.DMA,
        ],
    )(_emb_body)
    out = run(qi, pf, ov, wt24, pt_pad)
    return out.reshape(B, S, D)


# speedup vs baseline: 1.8770x; 1.5082x over previous
import functools

import jax
import jax.numpy as jnp
from jax import lax
from jax.experimental import pallas as pl
from jax.experimental.pallas import tpu as pltpu
from jax.experimental.pallas import tpu_sc as plsc

B, S = 4096, 200
N = B * S
WD = 60
PD = 4
D = WD + PD
L = 16
VW = 120
NW = 32
ROWS_PER_W = N // NW
CHUNK = 256
NCHUNK = ROWS_PER_W // CHUNK
NPAIR = NCHUNK // 2


def _emb_body(gi_hbm, pv_hbm, ov_hbm, wt_hbm, pt_hbm, out_hbm,
              gi0, gi1, pi0, pi1, ov0, ov1, win0, win1, comb0, comb1,
              ptv_v, sg0, sg1, so0, so1):
    wid = lax.axis_index("s") * 2 + lax.axis_index("c")
    base0 = wid * ROWS_PER_W
    pltpu.sync_copy(pt_hbm, ptv_v)
    io = lax.iota(jnp.int32, L)
    msk = io < (L - PD)

    gis = (gi0, gi1)
    pis = (pi0, pi1)
    ovs = (ov0, ov1)
    wins = (win0, win1)
    combs = (comb0, comb1)
    sgs = (sg0, sg1)
    sos = (so0, so1)

    def load_idx(i, k):
        base = base0 + i * CHUNK
        pltpu.sync_copy(gi_hbm.at[pl.ds(base, CHUNK)], gis[k])
        pltpu.sync_copy(pv_hbm.at[pl.ds(base, CHUNK)], pis[k])
        pltpu.sync_copy(ov_hbm.at[pl.ds(base, CHUNK)], ovs[k])

    def start_gather(k):
        return pltpu.async_copy(wt_hbm.at[gis[k]], wins[k], sgs[k])

    def wait_gather(k):
        pltpu.make_async_copy(wt_hbm.at[gis[k]], wins[k], sgs[k]).wait()

    def assemble(k):
        win_v = wins[k]
        comb_v = combs[k]

        def blk(b, c2):
            xo = ovs[k][pl.ds(b * L, L)]
            pv = pis[k][pl.ds(b * L, L)]
            for j in range(L):
                r = b * L + j
                off = xo[j]
                for m in range(3):
                    comb_v[pl.ds(D * r + m * L, L)] = \
                        win_v[r, pl.ds(off + m * L, L)]
                w3 = win_v[r, pl.ds(off + 3 * L, L)]
                comb_v[pl.ds(D * r + 3 * L, L)] = \
                    jnp.where(msk, w3, ptv_v[pv[j], :])
            return c2

        lax.fori_loop(0, CHUNK // L, blk, 0)

    def start_out(i, k):
        base = base0 + i * CHUNK
        return pltpu.async_copy(
            combs[k], out_hbm.at[pl.ds(D * base, D * CHUNK)], sos[k])

    def wait_out(i, k):
        base = base0 + i * CHUNK
        pltpu.make_async_copy(
            combs[k], out_hbm.at[pl.ds(D * base, D * CHUNK)], sos[k]).wait()

    load_idx(0, 0)
    start_gather(0)

    def pair(p, carry):
        for b in range(2):
            i = 2 * p + b
            nk = (b + 1) % 2

            @pl.when(i + 1 < NCHUNK)
            def _():
                load_idx(i + 1, nk)
                start_gather(nk)

            wait_gather(b)

            @pl.when(p > 0)
            def _():
                wait_out(i - 2, b)

            assemble(b)
            start_out(i, b)
        return carry

    lax.fori_loop(0, NPAIR, pair, 0)
    wait_out(NCHUNK - 2, 0)
    wait_out(NCHUNK - 1, 1)


def kernel(x, pos, word_table, pos_table):
    xf = x.reshape(N)
    pf = pos.reshape(N)
    wt120 = word_table.reshape(word_table.shape[0] * WD // VW, VW)
    gi = xf >> 1
    ov = WD * (xf & 1)
    pt_pad = jnp.zeros((pos_table.shape[0], L), pos_table.dtype)
    pt_pad = lax.dynamic_update_slice(pt_pad, pos_table, (0, L - PD))
    mesh = plsc.VectorSubcoreMesh(core_axis_name="c", subcore_axis_name="s")
    run = functools.partial(
        pl.kernel,
        mesh=mesh,
        compiler_params=pltpu.CompilerParams(use_tc_tiling_on_sc=False),
        out_type=jax.ShapeDtypeStruct((N * D,), jnp.float32),
        scratch_types=[
            pltpu.VMEM((CHUNK,), jnp.int32),
            pltpu.VMEM((CHUNK,), jnp.int32),
            pltpu.VMEM((CHUNK,), jnp.int32),
            pltpu.VMEM((CHUNK,), jnp.int32),
            pltpu.VMEM((CHUNK,), jnp.int32),
            pltpu.VMEM((CHUNK,), jnp.int32),
            pltpu.VMEM((CHUNK, VW), jnp.float32),
            pltpu.VMEM((CHUNK, VW), jnp.float32),
            pltpu.VMEM((CHUNK * D,), jnp.float32),
            pltpu.VMEM((CHUNK * D,), jnp.float32),
            pltpu.VMEM((pos_table.shape[0], L), jnp.float32),
            pltpu.SemaphoreType.DMA,
            pltpu.SemaphoreType.DMA,
            pltpu.SemaphoreType.DMA,
            pltpu.SemaphoreType.DMA,
        ],
    )(_emb_body)
    out = run(gi, pf, ov, wt120, pt_pad)
    return out.reshape(B, S, D)


# speedup vs baseline: 2.0666x; 1.1010x over previous
import functools

import jax
import jax.numpy as jnp
from jax import lax
from jax.experimental import pallas as pl
from jax.experimental.pallas import tpu as pltpu
from jax.experimental.pallas import tpu_sc as plsc

B, S = 4096, 200
N = B * S
WD = 60
PD = 4
D = WD + PD
L = 16
VW = 120
NW = 32
ROWS_PER_W = N // NW
CHUNK = 256
NCHUNK = ROWS_PER_W // CHUNK
NPAIR = NCHUNK // 2


def _emb_body(gi_hbm, pv_hbm, ov_hbm, wt_hbm, pt_hbm, out_hbm,
              gi0, gi1, pi0, pi1, ov0, ov1, win0, win1, comb0, comb1,
              ptv_v, sg0, sg1, so0, so1):
    wid = lax.axis_index("s") * 2 + lax.axis_index("c")
    base0 = wid * ROWS_PER_W
    pltpu.sync_copy(pt_hbm, ptv_v)
    io = lax.iota(jnp.int32, L)
    msk = io < (L - PD)

    gis = (gi0, gi1)
    pis = (pi0, pi1)
    ovs = (ov0, ov1)
    wins = (win0, win1)
    combs = (comb0, comb1)
    sgs = (sg0, sg1)
    sos = (so0, so1)

    def load_idx(i, k):
        base = base0 + i * CHUNK
        pltpu.sync_copy(gi_hbm.at[pl.ds(base, CHUNK)], gis[k])
        pltpu.sync_copy(pv_hbm.at[pl.ds(base, CHUNK)], pis[k])
        pltpu.sync_copy(ov_hbm.at[pl.ds(base, CHUNK)], ovs[k])

    def start_gather(k):
        return pltpu.async_copy(wt_hbm.at[gis[k]], wins[k], sgs[k])

    def wait_gather(k):
        pltpu.make_async_copy(wt_hbm.at[gis[k]], wins[k], sgs[k]).wait()

    def assemble(k):
        win_v = wins[k]
        comb_v = combs[k]

        @plsc.parallel_loop(0, CHUNK // L, 1, unroll=2)
        def blk(b):
            xo = ovs[k][pl.ds(b * L, L)]
            pv = pis[k][pl.ds(b * L, L)]
            for j in range(L):
                r = b * L + j
                off = xo[j]
                for m in range(3):
                    comb_v[pl.ds(D * r + m * L, L)] = \
                        win_v[r, pl.ds(off + m * L, L)]
                w3 = win_v[r, pl.ds(off + 3 * L, L)]
                comb_v[pl.ds(D * r + 3 * L, L)] = \
                    jnp.where(msk, w3, ptv_v[pv[j], :])

    def start_out(i, k):
        base = base0 + i * CHUNK
        return pltpu.async_copy(
            combs[k], out_hbm.at[pl.ds(D * base, D * CHUNK)], sos[k])

    def wait_out(i, k):
        base = base0 + i * CHUNK
        pltpu.make_async_copy(
            combs[k], out_hbm.at[pl.ds(D * base, D * CHUNK)], sos[k]).wait()

    load_idx(0, 0)
    start_gather(0)

    def pair(p, carry):
        for b in range(2):
            i = 2 * p + b
            nk = (b + 1) % 2

            @pl.when(i + 1 < NCHUNK)
            def _():
                load_idx(i + 1, nk)
                start_gather(nk)

            wait_gather(b)

            @pl.when(p > 0)
            def _():
                wait_out(i - 2, b)

            assemble(b)
            start_out(i, b)
        return carry

    lax.fori_loop(0, NPAIR, pair, 0)
    wait_out(NCHUNK - 2, 0)
    wait_out(NCHUNK - 1, 1)


def kernel(x, pos, word_table, pos_table):
    xf = x.reshape(N)
    pf = pos.reshape(N)
    wt120 = word_table.reshape(word_table.shape[0] * WD // VW, VW)
    gi = xf >> 1
    ov = WD * (xf & 1)
    pt_pad = jnp.zeros((pos_table.shape[0], L), pos_table.dtype)
    pt_pad = lax.dynamic_update_slice(pt_pad, pos_table, (0, L - PD))
    mesh = plsc.VectorSubcoreMesh(core_axis_name="c", subcore_axis_name="s")
    run = functools.partial(
        pl.kernel,
        mesh=mesh,
        compiler_params=pltpu.CompilerParams(use_tc_tiling_on_sc=False),
        out_type=jax.ShapeDtypeStruct((N * D,), jnp.float32),
        scratch_types=[
            pltpu.VMEM((CHUNK,), jnp.int32),
            pltpu.VMEM((CHUNK,), jnp.int32),
            pltpu.VMEM((CHUNK,), jnp.int32),
            pltpu.VMEM((CHUNK,), jnp.int32),
            pltpu.VMEM((CHUNK,), jnp.int32),
            pltpu.VMEM((CHUNK,), jnp.int32),
            pltpu.VMEM((CHUNK, VW), jnp.float32),
            pltpu.VMEM((CHUNK, VW), jnp.float32),
            pltpu.VMEM((CHUNK * D,), jnp.float32),
            pltpu.VMEM((CHUNK * D,), jnp.float32),
            pltpu.VMEM((pos_table.shape[0], L), jnp.float32),
            pltpu.SemaphoreType.DMA,
            pltpu.SemaphoreType.DMA,
            pltpu.SemaphoreType.DMA,
            pltpu.SemaphoreType.DMA,
        ],
    )(_emb_body)
    out = run(gi, pf, ov, wt120, pt_pad)
    return out.reshape(B, S, D)
